# baseline (speedup 1.0000x reference)
import functools

import jax
import jax.numpy as jnp
from jax import lax
from jax.experimental import pallas as pl
from jax.experimental.pallas import tpu as pltpu
from jax.experimental.pallas import tpu_sc as plsc

M = 32
NOUT = 10
TK = 6144

_INT_BIG = 1 << 30
_F_INF = float("inf")


def _score_body(q_ref, t_ref, w_ref, p_ref, a2_ref):
    t = t_ref[...]
    s = lax.dot_general(q_ref[...], t, (((1,), (1,)), ((), ())),
                        preferred_element_type=jnp.float32)
    rn = lax.dot_general(jnp.ones((1, t.shape[1]), jnp.float32), t * t,
                         (((1,), (1,)), ((), ())),
                         preferred_element_type=jnp.float32)
    p_ref[...] = rn - 2.0 * s

    @pl.when(pl.program_id(0) == 0)
    def _():
        a2_ref[...] = jnp.dot(q_ref[...], w_ref[...],
                              preferred_element_type=jnp.float32)


def _scores(queries, text_table, w_down):
    q, d = queries.shape
    k, _ = text_table.shape
    f = w_down.shape[1]
    kt = pl.cdiv(k, TK)
    return pl.pallas_call(
        _score_body,
        grid=(kt,),
        in_specs=[
            pl.BlockSpec((q, d), lambda i: (0, 0)),
            pl.BlockSpec((TK, d), lambda i: (i, 0)),
            pl.BlockSpec((d, f), lambda i: (0, 0)),
        ],
        out_specs=[
            pl.BlockSpec((q, TK), lambda i: (0, i)),
            pl.BlockSpec((q, f), lambda i: (0, 0)),
        ],
        out_shape=[
            jax.ShapeDtypeStruct((q, k), jnp.float32),
            jax.ShapeDtypeStruct((q, f), jnp.float32),
        ],
        compiler_params=pltpu.CompilerParams(
            dimension_semantics=("arbitrary",),
        ),
    )(queries, text_table, w_down)


def _sc_body(l, kcols, f,
             p_hbm, ids_hbm, table_hbm, ifac_hbm, iidx_hbm, q_hbm, a2_hbm,
             tid_hbm, tls_hbm, adj_hbm,
             ids_v, prow0_v, prow1_v, prox_v, mv_v, cid_v, cl_v, rows_v,
             qrow_v, loss_v, tid_v, tls_v, ifidx_v, ifrows_v, a2_v, adj_v,
             sem0, sem1, semg, sem2):
    wid = lax.axis_index("s") * 2 + lax.axis_index("c")
    qs_per_w = ids_v.shape[0]
    nvec = l // 16
    d = qrow_v.shape[0]
    q0 = wid * qs_per_w
    iota16 = lax.broadcasted_iota(jnp.int32, (16,), 0)

    def bcast_i(x):
        return lax.broadcast_in_dim(x, (16,), ())

    lane0 = iota16 == 0

    pltpu.sync_copy(ids_hbm.at[pl.ds(q0, qs_per_w)], ids_v)

    sems_p = [sem0, sem1]
    rows_p = [prow0_v, prow1_v]
    copies = [None, None]
    copies[0] = pltpu.async_copy(
        p_hbm.at[q0], rows_p[0], sems_p[0])
    for qi in range(qs_per_w):
        b = qi % 2
        if qi + 1 < qs_per_w:
            copies[(qi + 1) % 2] = pltpu.async_copy(
                p_hbm.at[q0 + qi + 1],
                rows_p[(qi + 1) % 2], sems_p[(qi + 1) % 2])
        copies[b].wait()
        prow = rows_p[b]

        def gbody(j2, _):
            for u in range(4):
                j = j2 * 4 + u
                idx = ids_v[qi, pl.ds(j * 16, 16)]
                v = plsc.load_gather(prow, [idx])
                prox_v[pl.ds(j * 16, 16)] = v
                plsc.store_scatter(mv_v, [bcast_i(j)], bcast_i(jnp.min(v)),
                                   mask=lane0)
            return 0

        lax.fori_loop(0, nvec // 4, gbody, 0)

        def sbody(r, _):
            mv0 = mv_v[pl.ds(0, 16)]
            mv1 = mv_v[pl.ds(16, 16)]
            m = jnp.min(jnp.minimum(mv0, mv1))
            c0 = jnp.where(mv0 == m, iota16, _INT_BIG)
            c1 = jnp.where(mv1 == m, iota16 + 16, _INT_BIG)
            j = jnp.minimum(jnp.min(c0), jnp.min(c1))
            v = prox_v[pl.ds(j * 16, 16)]
            lane = jnp.min(jnp.where(v == m, iota16, _INT_BIG))
            idvec = ids_v[qi, pl.ds(j * 16, 16)]
            cid = jnp.min(jnp.where(iota16 == lane, idvec, _INT_BIG))
            pos = bcast_i(qi * M + r)
            plsc.store_scatter(cid_v, [pos], bcast_i(cid), mask=lane0)
            plsc.store_scatter(cl_v, [pos], bcast_i(j * 16 + lane), mask=lane0)
            v2 = jnp.where(iota16 == lane, _F_INF, v)
            prox_v[pl.ds(j * 16, 16)] = v2
            plsc.store_scatter(mv_v, [bcast_i(j)], bcast_i(jnp.min(v2)),
                               mask=lane0)
            return 0

        lax.fori_loop(0, M, sbody, 0)

        rcopy = pltpu.async_copy(table_hbm.at[cid_v.at[pl.ds(qi * M, M)]],
                                 rows_v, semg)
        pltpu.sync_copy(q_hbm.at[q0 + qi], qrow_v)
        rcopy.wait()

        def lbody(c, _):
            def bbody(bb, acc):
                for u in range(8):
                    off = bb * 128 + u * 16
                    dd = (qrow_v[pl.ds(off, 16)]
                          - rows_v[c, pl.ds(off, 16)])
                    acc = acc + dd * dd
                return acc

            acc = lax.fori_loop(0, d // 128, bbody,
                                jnp.zeros((16,), jnp.float32))
            plsc.store_scatter(loss_v, [bcast_i(c)], bcast_i(jnp.sum(acc)),
                               mask=lane0)
            return 0

        lax.fori_loop(0, M, lbody, 0)

        cid0 = cid_v[pl.ds(qi * M, 16)]
        cid1 = cid_v[pl.ds(qi * M + 16, 16)]
        cl0 = cl_v[pl.ds(qi * M, 16)]
        cl1 = cl_v[pl.ds(qi * M + 16, 16)]

        def fbody(r, _):
            ls0 = loss_v[pl.ds(0, 16)]
            ls1 = loss_v[pl.ds(16, 16)]
            m = jnp.min(jnp.minimum(ls0, ls1))
            lsel = jnp.minimum(
                jnp.min(jnp.where(ls0 == m, cl0, _INT_BIG)),
                jnp.min(jnp.where(ls1 == m, cl1, _INT_BIG)))
            hit0 = (ls0 == m) & (cl0 == lsel)
            hit1 = (ls1 == m) & (cl1 == lsel)
            oid = jnp.minimum(
                jnp.min(jnp.where(hit0, cid0, _INT_BIG)),
                jnp.min(jnp.where(hit1, cid1, _INT_BIG)))
            pos = bcast_i(qi * NOUT + r)
            plsc.store_scatter(tid_v, [pos], bcast_i(oid), mask=lane0)
            plsc.store_scatter(tls_v, [pos], bcast_i(m), mask=lane0)
            loss_v[pl.ds(0, 16)] = jnp.where(hit0, _F_INF, ls0)
            loss_v[pl.ds(16, 16)] = jnp.where(hit1, _F_INF, ls1)
            return 0

        lax.fori_loop(0, NOUT, fbody, 0)

    pltpu.sync_copy(tid_v, tid_hbm.at[pl.ds(q0 * NOUT, qs_per_w * NOUT)])
    pltpu.sync_copy(tls_v, tls_hbm.at[pl.ds(q0 * NOUT, qs_per_w * NOUT)])

    nq = adj_hbm.shape[0]
    per_i = 8
    nworkers_i = nq // per_i

    @pl.when(wid < nworkers_i)
    def _():
        pltpu.sync_copy(iidx_hbm.at[pl.ds(wid * per_i, per_i)], ifidx_v)
        pltpu.async_copy(ifac_hbm.at[ifidx_v], ifrows_v, sem2).wait()
        pltpu.sync_copy(a2_hbm.at[pl.ds(wid * per_i, per_i)], a2_v)
        for qq in range(per_i):
            acc = jnp.zeros((16,), jnp.float32)
            for bb in range(f // 16):
                acc = acc + (a2_v[qq, pl.ds(bb * 16, 16)]
                             * ifrows_v[qq, pl.ds(bb * 16, 16)])
            plsc.store_scatter(adj_v, [bcast_i(qq)], bcast_i(jnp.sum(acc)),
                               mask=lane0)
        pltpu.sync_copy(adj_v, adj_hbm.at[pl.ds(wid * per_i, per_i)])


def _sc_stage(p2d, ids2d, text_table, ifac_pad, item_idx, queries, a2,
              kcols, l):
    nq, _ = ids2d.shape
    qs_per_w = nq // 32
    d = text_table.shape[1]
    f = a2.shape[1]
    fpad = ifac_pad.shape[1]
    mesh = plsc.VectorSubcoreMesh(core_axis_name="c", subcore_axis_name="s")
    kern = pl.kernel(
        functools.partial(_sc_body, l, kcols, f),
        out_type=[
            jax.ShapeDtypeStruct((nq * NOUT,), jnp.int32),
            jax.ShapeDtypeStruct((nq * NOUT,), jnp.float32),
            jax.ShapeDtypeStruct((nq,), jnp.float32),
        ],
        mesh=mesh,
        scratch_types=[
            pltpu.VMEM((qs_per_w, l), jnp.int32),
            pltpu.VMEM((kcols,), jnp.float32),
            pltpu.VMEM((kcols,), jnp.float32),
            pltpu.VMEM((l,), jnp.float32),
            pltpu.VMEM((l // 16,), jnp.float32),
            pltpu.VMEM((qs_per_w * M,), jnp.int32),
            pltpu.VMEM((qs_per_w * M,), jnp.int32),
            pltpu.VMEM((M, d), jnp.float32),
            pltpu.VMEM((d,), jnp.float32),
            pltpu.VMEM((M,), jnp.float32),
            pltpu.VMEM((qs_per_w * NOUT,), jnp.int32),
            pltpu.VMEM((qs_per_w * NOUT,), jnp.float32),
            pltpu.VMEM((8,), jnp.int32),
            pltpu.VMEM((8, fpad), jnp.float32),
            pltpu.VMEM((8, f), jnp.float32),
            pltpu.VMEM((8,), jnp.float32),
            pltpu.SemaphoreType.DMA,
            pltpu.SemaphoreType.DMA,
            pltpu.SemaphoreType.DMA,
            pltpu.SemaphoreType.DMA,
        ],
        compiler_params=pltpu.CompilerParams(needs_layout_passes=False),
    )
    return kern(p2d, ids2d, text_table, ifac_pad, item_idx, queries, a2)


def kernel(queries, text_table, W_downproj, item_factor, input_ids,
           item_idx, N):
    q, d = queries.shape
    k = text_table.shape[0]
    l = input_ids.shape[1]

    p, a2 = _scores(queries, text_table, W_downproj)
    f = item_factor.shape[1]
    ifac_pad = jnp.pad(item_factor, ((0, 0), (0, 128 - f)))
    tid_flat, tls_flat, adj = _sc_stage(
        p, input_ids.astype(jnp.int32), text_table, ifac_pad,
        item_idx.astype(jnp.int32), queries, a2, k, l)
    return tid_flat.reshape(q, NOUT), tls_flat.reshape(q, NOUT), adj


# speedup vs baseline: 1.0108x; 1.0108x over previous
import functools

import jax
import jax.numpy as jnp
from jax import lax
from jax.experimental import pallas as pl
from jax.experimental.pallas import tpu as pltpu
from jax.experimental.pallas import tpu_sc as plsc

M = 32
NOUT = 10
TK = 4096

_INT_BIG = 1 << 30
_F_INF = float("inf")


def _score_body(q_ref, t_ref, w_ref, p_ref, a2_ref):
    t = t_ref[...]
    s = lax.dot_general(q_ref[...], t, (((1,), (1,)), ((), ())),
                        preferred_element_type=jnp.float32)
    rn = lax.dot_general(jnp.ones((1, t.shape[1]), jnp.float32), t * t,
                         (((1,), (1,)), ((), ())),
                         preferred_element_type=jnp.float32)
    p_ref[...] = rn - 2.0 * s

    @pl.when(pl.program_id(0) == 0)
    def _():
        a2_ref[...] = jnp.dot(q_ref[...], w_ref[...],
                              preferred_element_type=jnp.float32)


def _scores(queries, text_table, w_down):
    q, d = queries.shape
    k, _ = text_table.shape
    f = w_down.shape[1]
    kt = pl.cdiv(k, TK)
    return pl.pallas_call(
        _score_body,
        grid=(kt,),
        in_specs=[
            pl.BlockSpec((q, d), lambda i: (0, 0)),
            pl.BlockSpec((TK, d), lambda i: (i, 0)),
            pl.BlockSpec((d, f), lambda i: (0, 0)),
        ],
        out_specs=[
            pl.BlockSpec((q, TK), lambda i: (0, i)),
            pl.BlockSpec((q, f), lambda i: (0, 0)),
        ],
        out_shape=[
            jax.ShapeDtypeStruct((q, k), jnp.float32),
            jax.ShapeDtypeStruct((q, f), jnp.float32),
        ],
        compiler_params=pltpu.CompilerParams(
            dimension_semantics=("arbitrary",),
        ),
    )(queries, text_table, w_down)


def _sc_body(l, kcols, f,
             p_hbm, ids_hbm, table_hbm, ifac_hbm, iidx_hbm, q_hbm, a2_hbm,
             tid_hbm, tls_hbm, adj_hbm,
             ids_v, prow0_v, prow1_v, prox_v, mv_v, cid_v, cl_v, rows_v,
             qrow_v, loss_v, tid_v, tls_v, ifidx_v, ifrows_v, a2_v, adj_v,
             sem0, sem1, semg, sem2):
    wid = lax.axis_index("s") * 2 + lax.axis_index("c")
    qs_per_w = ids_v.shape[0]
    nvec = l // 16
    d = qrow_v.shape[0]
    q0 = wid * qs_per_w
    iota16 = lax.broadcasted_iota(jnp.int32, (16,), 0)

    def bcast_i(x):
        return lax.broadcast_in_dim(x, (16,), ())

    lane0 = iota16 == 0

    pltpu.sync_copy(ids_hbm.at[pl.ds(q0, qs_per_w)], ids_v)

    sems_p = [sem0, sem1]
    rows_p = [prow0_v, prow1_v]
    copies = [None, None]
    copies[0] = pltpu.async_copy(
        p_hbm.at[q0], rows_p[0], sems_p[0])
    for qi in range(qs_per_w):
        b = qi % 2
        if qi + 1 < qs_per_w:
            copies[(qi + 1) % 2] = pltpu.async_copy(
                p_hbm.at[q0 + qi + 1],
                rows_p[(qi + 1) % 2], sems_p[(qi + 1) % 2])
        copies[b].wait()
        prow = rows_p[b]

        def gbody(j2, _):
            for u in range(4):
                j = j2 * 4 + u
                idx = ids_v[qi, pl.ds(j * 16, 16)]
                v = plsc.load_gather(prow, [idx])
                prox_v[pl.ds(j * 16, 16)] = v
                plsc.store_scatter(mv_v, [bcast_i(j)], bcast_i(jnp.min(v)),
                                   mask=lane0)
            return 0

        lax.fori_loop(0, nvec // 4, gbody, 0)

        def sbody(r, _):
            mv0 = mv_v[pl.ds(0, 16)]
            mv1 = mv_v[pl.ds(16, 16)]
            m = jnp.min(jnp.minimum(mv0, mv1))
            c0 = jnp.where(mv0 == m, iota16, _INT_BIG)
            c1 = jnp.where(mv1 == m, iota16 + 16, _INT_BIG)
            j = jnp.minimum(jnp.min(c0), jnp.min(c1))
            v = prox_v[pl.ds(j * 16, 16)]
            lane = jnp.min(jnp.where(v == m, iota16, _INT_BIG))
            idvec = ids_v[qi, pl.ds(j * 16, 16)]
            cid = jnp.min(jnp.where(iota16 == lane, idvec, _INT_BIG))
            pos = bcast_i(qi * M + r)
            plsc.store_scatter(cid_v, [pos], bcast_i(cid), mask=lane0)
            plsc.store_scatter(cl_v, [pos], bcast_i(j * 16 + lane), mask=lane0)
            v2 = jnp.where(iota16 == lane, _F_INF, v)
            prox_v[pl.ds(j * 16, 16)] = v2
            plsc.store_scatter(mv_v, [bcast_i(j)], bcast_i(jnp.min(v2)),
                               mask=lane0)
            return 0

        lax.fori_loop(0, M, sbody, 0)

        rcopy = pltpu.async_copy(table_hbm.at[cid_v.at[pl.ds(qi * M, M)]],
                                 rows_v, semg)
        pltpu.sync_copy(q_hbm.at[q0 + qi], qrow_v)
        rcopy.wait()

        def lbody(c, _):
            def bbody(bb, acc):
                for u in range(8):
                    off = bb * 128 + u * 16
                    dd = (qrow_v[pl.ds(off, 16)]
                          - rows_v[c, pl.ds(off, 16)])
                    acc = acc + dd * dd
                return acc

            acc = lax.fori_loop(0, d // 128, bbody,
                                jnp.zeros((16,), jnp.float32))
            plsc.store_scatter(loss_v, [bcast_i(c)], bcast_i(jnp.sum(acc)),
                               mask=lane0)
            return 0

        lax.fori_loop(0, M, lbody, 0)

        cid0 = cid_v[pl.ds(qi * M, 16)]
        cid1 = cid_v[pl.ds(qi * M + 16, 16)]
        cl0 = cl_v[pl.ds(qi * M, 16)]
        cl1 = cl_v[pl.ds(qi * M + 16, 16)]

        def fbody(r, _):
            ls0 = loss_v[pl.ds(0, 16)]
            ls1 = loss_v[pl.ds(16, 16)]
            m = jnp.min(jnp.minimum(ls0, ls1))
            lsel = jnp.minimum(
                jnp.min(jnp.where(ls0 == m, cl0, _INT_BIG)),
                jnp.min(jnp.where(ls1 == m, cl1, _INT_BIG)))
            hit0 = (ls0 == m) & (cl0 == lsel)
            hit1 = (ls1 == m) & (cl1 == lsel)
            oid = jnp.minimum(
                jnp.min(jnp.where(hit0, cid0, _INT_BIG)),
                jnp.min(jnp.where(hit1, cid1, _INT_BIG)))
            pos = bcast_i(qi * NOUT + r)
            plsc.store_scatter(tid_v, [pos], bcast_i(oid), mask=lane0)
            plsc.store_scatter(tls_v, [pos], bcast_i(m), mask=lane0)
            loss_v[pl.ds(0, 16)] = jnp.where(hit0, _F_INF, ls0)
            loss_v[pl.ds(16, 16)] = jnp.where(hit1, _F_INF, ls1)
            return 0

        lax.fori_loop(0, NOUT, fbody, 0)

    pltpu.sync_copy(tid_v, tid_hbm.at[pl.ds(q0 * NOUT, qs_per_w * NOUT)])
    pltpu.sync_copy(tls_v, tls_hbm.at[pl.ds(q0 * NOUT, qs_per_w * NOUT)])

    nq = adj_hbm.shape[0]
    per_i = 8
    nworkers_i = nq // per_i

    @pl.when(wid < nworkers_i)
    def _():
        pltpu.sync_copy(iidx_hbm.at[pl.ds(wid * per_i, per_i)], ifidx_v)
        pltpu.async_copy(ifac_hbm.at[ifidx_v], ifrows_v, sem2).wait()
        pltpu.sync_copy(a2_hbm.at[pl.ds(wid * per_i, per_i)], a2_v)
        for qq in range(per_i):
            acc = jnp.zeros((16,), jnp.float32)
            for bb in range(f // 16):
                acc = acc + (a2_v[qq, pl.ds(bb * 16, 16)]
                             * ifrows_v[qq, pl.ds(bb * 16, 16)])
            plsc.store_scatter(adj_v, [bcast_i(qq)], bcast_i(jnp.sum(acc)),
                               mask=lane0)
        pltpu.sync_copy(adj_v, adj_hbm.at[pl.ds(wid * per_i, per_i)])


def _sc_stage(p2d, ids2d, text_table, ifac_pad, item_idx, queries, a2,
              kcols, l):
    nq, _ = ids2d.shape
    qs_per_w = nq // 32
    d = text_table.shape[1]
    f = a2.shape[1]
    fpad = ifac_pad.shape[1]
    mesh = plsc.VectorSubcoreMesh(core_axis_name="c", subcore_axis_name="s")
    kern = pl.kernel(
        functools.partial(_sc_body, l, kcols, f),
        out_type=[
            jax.ShapeDtypeStruct((nq * NOUT,), jnp.int32),
            jax.ShapeDtypeStruct((nq * NOUT,), jnp.float32),
            jax.ShapeDtypeStruct((nq,), jnp.float32),
        ],
        mesh=mesh,
        scratch_types=[
            pltpu.VMEM((qs_per_w, l), jnp.int32),
            pltpu.VMEM((kcols,), jnp.float32),
            pltpu.VMEM((kcols,), jnp.float32),
            pltpu.VMEM((l,), jnp.float32),
            pltpu.VMEM((l // 16,), jnp.float32),
            pltpu.VMEM((qs_per_w * M,), jnp.int32),
            pltpu.VMEM((qs_per_w * M,), jnp.int32),
            pltpu.VMEM((M, d), jnp.float32),
            pltpu.VMEM((d,), jnp.float32),
            pltpu.VMEM((M,), jnp.float32),
            pltpu.VMEM((qs_per_w * NOUT,), jnp.int32),
            pltpu.VMEM((qs_per_w * NOUT,), jnp.float32),
            pltpu.VMEM((8,), jnp.int32),
            pltpu.VMEM((8, fpad), jnp.float32),
            pltpu.VMEM((8, f), jnp.float32),
            pltpu.VMEM((8,), jnp.float32),
            pltpu.SemaphoreType.DMA,
            pltpu.SemaphoreType.DMA,
            pltpu.SemaphoreType.DMA,
            pltpu.SemaphoreType.DMA,
        ],
        compiler_params=pltpu.CompilerParams(needs_layout_passes=False),
    )
    return kern(p2d, ids2d, text_table, ifac_pad, item_idx, queries, a2)


def kernel(queries, text_table, W_downproj, item_factor, input_ids,
           item_idx, N):
    q, d = queries.shape
    k = text_table.shape[0]
    l = input_ids.shape[1]

    p, a2 = _scores(queries, text_table, W_downproj)
    f = item_factor.shape[1]
    ifac_pad = jnp.pad(item_factor, ((0, 0), (0, 128 - f)))
    tid_flat, tls_flat, adj = _sc_stage(
        p, input_ids.astype(jnp.int32), text_table, ifac_pad,
        item_idx.astype(jnp.int32), queries, a2, k, l)
    return tid_flat.reshape(q, NOUT), tls_flat.reshape(q, NOUT), adj
